# baseline (speedup 1.0000x reference)
import functools

import jax
import jax.numpy as jnp
from jax import lax
from jax.experimental import pallas as pl
from jax.experimental.pallas import tpu as pltpu
from jax.experimental.pallas import tpu_sc as plsc

NUM_VECTORS = 1024
LATENT_DIM = 256
B, C, H, W = 8, 256, 32, 32
HW = H * W


def _vq_kernel(x_ref, xx_ref, ee_ref, e_ref, ehi_ref, emids_ref,
               xt_ref, xqd_ref, idx_ref):
    xb = x_ref[0]
    xt_ref[0] = xb.T

    e = e_ref[...]
    mm = jax.lax.dot_general(e, xb, (((1,), (0,)), ((), ())),
                             preferred_element_type=jnp.float32)
    xx = xx_ref[0]
    ee = ee_ref[...]
    d = (xx + ee) - 2.0 * mm

    dmin = jnp.min(d, axis=0, keepdims=True)
    iota = jax.lax.broadcasted_iota(jnp.int32, (NUM_VECTORS, HW), 0)
    cand = jnp.where(d == dmin, iota, jnp.int32(NUM_VECTORS))
    idx = jnp.min(cand, axis=0, keepdims=True)
    idx_ref[0] = idx

    onehot = (iota == idx).astype(jnp.bfloat16)
    cdims = (((0,), (0,)), ((), ()))
    xqd = (jax.lax.dot_general(ehi_ref[...], onehot, cdims,
                               preferred_element_type=jnp.float32)
           + jax.lax.dot_general(emids_ref[...], onehot, cdims,
                                 preferred_element_type=jnp.float32)
           * jnp.float32(2.0 ** -9))
    xqd_ref[0] = xb + (xqd - xb)


def _make_sc_gather():
    info = plsc.get_sparse_core_info()
    nw = info.num_cores * info.num_subcores
    b_per_w = (B * HW) // nw
    mesh = plsc.VectorSubcoreMesh(core_axis_name="c", subcore_axis_name="s")

    @functools.partial(
        pl.kernel,
        out_type=jax.ShapeDtypeStruct((B * HW, LATENT_DIM), jnp.float32),
        mesh=mesh,
        scratch_types=[
            pltpu.VMEM((b_per_w,), jnp.int32),
            pltpu.VMEM((b_per_w, LATENT_DIM), jnp.float32),
            pltpu.SemaphoreType.DMA,
        ],
    )
    def gather(table_hbm, idx_hbm, out_hbm, idx_v, rows_v, sem):
        wid = lax.axis_index("s") * info.num_cores + lax.axis_index("c")
        base = wid * b_per_w
        pltpu.sync_copy(idx_hbm.at[pl.ds(base, b_per_w)], idx_v)
        pltpu.async_copy(table_hbm.at[idx_v], rows_v, sem).wait()
        pltpu.sync_copy(rows_v, out_hbm.at[pl.ds(base, b_per_w)])

    return gather


_sc_gather = _make_sc_gather()


def kernel(x, embedding_weight):
    x_t_outer = jnp.transpose(x, (0, 2, 3, 1))
    x_flat = x_t_outer.reshape(-1, x_t_outer.shape[-1])
    xx = jnp.sum(x_flat ** 2, axis=1, keepdims=True)
    ee = jnp.sum(embedding_weight ** 2, axis=1)

    u = jax.lax.bitcast_convert_type(embedding_weight, jnp.uint32)
    e_hi_f = jax.lax.bitcast_convert_type(u & jnp.uint32(0xFFFF0000),
                                          jnp.float32)
    e_hi = e_hi_f.astype(jnp.bfloat16)
    r = embedding_weight - e_hi_f
    e_mid_s = (r * jnp.float32(2.0 ** 9)).astype(jnp.bfloat16)

    x3 = x.reshape(B, C, HW)
    xx3 = xx.reshape(B, 1, HW)
    ee2 = ee.reshape(NUM_VECTORS, 1)

    full = lambda b: (0, 0)
    xt, xqd, idx = pl.pallas_call(
        _vq_kernel,
        grid=(B,),
        in_specs=[
            pl.BlockSpec((1, C, HW), lambda b: (b, 0, 0)),
            pl.BlockSpec((1, 1, HW), lambda b: (b, 0, 0)),
            pl.BlockSpec((NUM_VECTORS, 1), full),
            pl.BlockSpec((NUM_VECTORS, LATENT_DIM), full),
            pl.BlockSpec((NUM_VECTORS, LATENT_DIM), full),
            pl.BlockSpec((NUM_VECTORS, LATENT_DIM), full),
        ],
        out_specs=[
            pl.BlockSpec((1, HW, C), lambda b: (b, 0, 0)),
            pl.BlockSpec((1, C, HW), lambda b: (b, 0, 0)),
            pl.BlockSpec((1, 1, HW), lambda b: (b, 0, 0)),
        ],
        out_shape=[
            jax.ShapeDtypeStruct((B, HW, C), jnp.float32),
            jax.ShapeDtypeStruct((B, C, HW), jnp.float32),
            jax.ShapeDtypeStruct((B, 1, HW), jnp.int32),
        ],
    )(x3, xx3, ee2, embedding_weight, e_hi, e_mid_s)

    xq = _sc_gather(embedding_weight, idx.reshape(B * HW))

    x_t = xt.reshape(B, H, W, C)
    x_q = xq.reshape(B, H, W, C)
    x_q_detach = xqd.reshape(B, C, H, W)
    return (x_q_detach, x_q, x_t)


# speedup vs baseline: 1.3922x; 1.3922x over previous
import jax
import jax.numpy as jnp
from jax.experimental import pallas as pl
from jax.experimental.pallas import tpu as pltpu

NUM_VECTORS = 1024
LATENT_DIM = 256
B, C, H, W = 8, 256, 32, 32
HW = H * W
TT = 1024
NT = HW // TT


def _vq_kernel(x_ref, xx_ref, ee_ref, e_ref, ehi_ref, emids_ref,
               xt_ref, xq_ref, xqd_ref):
    xb = x_ref[0]
    xt_ref[0] = xb.T

    e = e_ref[...]
    mm = jax.lax.dot_general(e, xb, (((1,), (0,)), ((), ())),
                             preferred_element_type=jnp.float32)
    xx = xx_ref[0]
    ee = ee_ref[...]
    d = (xx + ee) - 2.0 * mm

    dmin = jnp.min(d, axis=0, keepdims=True)
    iota = jax.lax.broadcasted_iota(jnp.int32, (NUM_VECTORS, TT), 0)
    cand = jnp.where(d == dmin, iota, jnp.int32(NUM_VECTORS))
    idx = jnp.min(cand, axis=0, keepdims=True)

    onehot = (iota == idx).astype(jnp.bfloat16)
    cdims = (((0,), (0,)), ((), ()))
    xqd = (jax.lax.dot_general(ehi_ref[...], onehot, cdims,
                               preferred_element_type=jnp.float32)
           + jax.lax.dot_general(emids_ref[...], onehot, cdims,
                                 preferred_element_type=jnp.float32)
           * jnp.float32(2.0 ** -9))
    xqd_ref[0] = xb + (xqd - xb)
    xq_ref[0] = xqd.T


def kernel(x, embedding_weight):
    x_t_outer = jnp.transpose(x, (0, 2, 3, 1))
    x_flat = x_t_outer.reshape(-1, x_t_outer.shape[-1])
    xx = jnp.sum(x_flat ** 2, axis=1, keepdims=True)
    ee = jnp.sum(embedding_weight ** 2, axis=1)

    u = jax.lax.bitcast_convert_type(embedding_weight, jnp.uint32)
    e_hi_f = jax.lax.bitcast_convert_type(u & jnp.uint32(0xFFFF0000),
                                          jnp.float32)
    e_hi = e_hi_f.astype(jnp.bfloat16)
    r = embedding_weight - e_hi_f
    e_mid_s = (r * jnp.float32(2.0 ** 9)).astype(jnp.bfloat16)

    x3 = x.reshape(B, C, HW)
    xx3 = xx.reshape(B, 1, HW)
    ee2 = ee.reshape(NUM_VECTORS, 1)

    full = lambda b, t: (0, 0)
    xt, xq, xqd = pl.pallas_call(
        _vq_kernel,
        grid=(B, NT),
        compiler_params=pltpu.CompilerParams(
            dimension_semantics=("parallel", "parallel")),
        in_specs=[
            pl.BlockSpec((1, C, TT), lambda b, t: (b, 0, t)),
            pl.BlockSpec((1, 1, TT), lambda b, t: (b, 0, t)),
            pl.BlockSpec((NUM_VECTORS, 1), full),
            pl.BlockSpec((NUM_VECTORS, LATENT_DIM), full),
            pl.BlockSpec((NUM_VECTORS, LATENT_DIM), full),
            pl.BlockSpec((NUM_VECTORS, LATENT_DIM), full),
        ],
        out_specs=[
            pl.BlockSpec((1, TT, C), lambda b, t: (b, t, 0)),
            pl.BlockSpec((1, TT, C), lambda b, t: (b, t, 0)),
            pl.BlockSpec((1, C, TT), lambda b, t: (b, 0, t)),
        ],
        out_shape=[
            jax.ShapeDtypeStruct((B, HW, C), jnp.float32),
            jax.ShapeDtypeStruct((B, HW, C), jnp.float32),
            jax.ShapeDtypeStruct((B, C, HW), jnp.float32),
        ],
    )(x3, xx3, ee2, embedding_weight, e_hi, e_mid_s)

    x_t = xt.reshape(B, H, W, C)
    x_q = xq.reshape(B, H, W, C)
    x_q_detach = xqd.reshape(B, C, H, W)
    return (x_q_detach, x_q, x_t)


# speedup vs baseline: 1.3982x; 1.0043x over previous
import jax
import jax.numpy as jnp
from jax.experimental import pallas as pl
from jax.experimental.pallas import tpu as pltpu

NUM_VECTORS = 1024
LATENT_DIM = 256
B, C, H, W = 8, 256, 32, 32
HW = H * W
TT = 1024
NT = HW // TT


def _vq_kernel(x_ref, xx_ref, ee_ref, e_ref, ehi_ref, emids_ref,
               xt_ref, xq_ref, xqd_ref):
    xb = x_ref[0]
    xt_ref[0] = xb.T

    e = e_ref[...]
    mm = jax.lax.dot_general(e, xb, (((1,), (0,)), ((), ())),
                             preferred_element_type=jnp.float32)
    xx = xx_ref[0]
    ee = ee_ref[...]
    d = (xx + ee) - 2.0 * mm

    dmin = jnp.min(d, axis=0, keepdims=True)
    iota = jax.lax.broadcasted_iota(jnp.int32, (NUM_VECTORS, TT), 0
                                    ).astype(jnp.float32)
    cand = jnp.where(d == dmin, iota, jnp.float32(NUM_VECTORS))
    idx = jnp.min(cand, axis=0, keepdims=True)

    onehot = (iota == idx).astype(jnp.bfloat16)
    cdims = (((0,), (0,)), ((), ()))
    xqd = (jax.lax.dot_general(ehi_ref[...], onehot, cdims,
                               preferred_element_type=jnp.float32)
           + jax.lax.dot_general(emids_ref[...], onehot, cdims,
                                 preferred_element_type=jnp.float32)
           * jnp.float32(2.0 ** -9))
    xqd_ref[0] = xb + (xqd - xb)
    xq_ref[0] = xqd.T


def kernel(x, embedding_weight):
    x_t_outer = jnp.transpose(x, (0, 2, 3, 1))
    x_flat = x_t_outer.reshape(-1, x_t_outer.shape[-1])
    xx = jnp.sum(x_flat ** 2, axis=1, keepdims=True)
    ee = jnp.sum(embedding_weight ** 2, axis=1)

    u = jax.lax.bitcast_convert_type(embedding_weight, jnp.uint32)
    e_hi_f = jax.lax.bitcast_convert_type(u & jnp.uint32(0xFFFF0000),
                                          jnp.float32)
    e_hi = e_hi_f.astype(jnp.bfloat16)
    r = embedding_weight - e_hi_f
    e_mid_s = (r * jnp.float32(2.0 ** 9)).astype(jnp.bfloat16)

    x3 = x.reshape(B, C, HW)
    xx3 = xx.reshape(B, 1, HW)
    ee2 = ee.reshape(NUM_VECTORS, 1)

    full = lambda b, t: (0, 0)
    xt, xq, xqd = pl.pallas_call(
        _vq_kernel,
        grid=(B, NT),
        compiler_params=pltpu.CompilerParams(
            dimension_semantics=("parallel", "parallel")),
        in_specs=[
            pl.BlockSpec((1, C, TT), lambda b, t: (b, 0, t)),
            pl.BlockSpec((1, 1, TT), lambda b, t: (b, 0, t)),
            pl.BlockSpec((NUM_VECTORS, 1), full),
            pl.BlockSpec((NUM_VECTORS, LATENT_DIM), full),
            pl.BlockSpec((NUM_VECTORS, LATENT_DIM), full),
            pl.BlockSpec((NUM_VECTORS, LATENT_DIM), full),
        ],
        out_specs=[
            pl.BlockSpec((1, TT, C), lambda b, t: (b, t, 0)),
            pl.BlockSpec((1, TT, C), lambda b, t: (b, t, 0)),
            pl.BlockSpec((1, C, TT), lambda b, t: (b, 0, t)),
        ],
        out_shape=[
            jax.ShapeDtypeStruct((B, HW, C), jnp.float32),
            jax.ShapeDtypeStruct((B, HW, C), jnp.float32),
            jax.ShapeDtypeStruct((B, C, HW), jnp.float32),
        ],
    )(x3, xx3, ee2, embedding_weight, e_hi, e_mid_s)

    x_t = xt.reshape(B, H, W, C)
    x_q = xq.reshape(B, H, W, C)
    x_q_detach = xqd.reshape(B, C, H, W)
    return (x_q_detach, x_q, x_t)


# speedup vs baseline: 1.4046x; 1.0046x over previous
import jax
import jax.numpy as jnp
from jax.experimental import pallas as pl
from jax.experimental.pallas import tpu as pltpu

NUM_VECTORS = 1024
LATENT_DIM = 256
B, C, H, W = 8, 256, 32, 32
HW = H * W
BB = 2
NB = B // BB


def _vq_kernel(x_ref, xx_ref, ee_ref, e_ref, ehi_ref, emids_ref,
               xt_ref, xq_ref, xqd_ref):
    e = e_ref[...]
    ee = ee_ref[...]
    ehi = ehi_ref[...]
    emids = emids_ref[...]
    for i in range(BB):
        xb = x_ref[i]
        xt_ref[i] = xb.T

        mm = jax.lax.dot_general(e, xb, (((1,), (0,)), ((), ())),
                                 preferred_element_type=jnp.float32)
        xx = xx_ref[i]
        d = (xx + ee) - 2.0 * mm

        dmin = jnp.min(d, axis=0, keepdims=True)
        iota = jax.lax.broadcasted_iota(jnp.int32, (NUM_VECTORS, HW), 0
                                        ).astype(jnp.float32)
        cand = jnp.where(d == dmin, iota, jnp.float32(NUM_VECTORS))
        idx = jnp.min(cand, axis=0, keepdims=True)

        onehot = (iota == idx).astype(jnp.bfloat16)
        cdims = (((0,), (0,)), ((), ()))
        xqd = (jax.lax.dot_general(ehi, onehot, cdims,
                                   preferred_element_type=jnp.float32)
               + jax.lax.dot_general(emids, onehot, cdims,
                                     preferred_element_type=jnp.float32)
               * jnp.float32(2.0 ** -9))
        xqd_ref[i] = xb + (xqd - xb)
        xq_ref[i] = xqd.T


def kernel(x, embedding_weight):
    x_t_outer = jnp.transpose(x, (0, 2, 3, 1))
    x_flat = x_t_outer.reshape(-1, x_t_outer.shape[-1])
    xx = jnp.sum(x_flat ** 2, axis=1, keepdims=True)
    ee = jnp.sum(embedding_weight ** 2, axis=1)

    u = jax.lax.bitcast_convert_type(embedding_weight, jnp.uint32)
    e_hi_f = jax.lax.bitcast_convert_type(u & jnp.uint32(0xFFFF0000),
                                          jnp.float32)
    e_hi = e_hi_f.astype(jnp.bfloat16)
    r = embedding_weight - e_hi_f
    e_mid_s = (r * jnp.float32(2.0 ** 9)).astype(jnp.bfloat16)

    x3 = x.reshape(B, C, HW)
    xx3 = xx.reshape(B, 1, HW)
    ee2 = ee.reshape(NUM_VECTORS, 1)

    full = lambda b: (0, 0)
    xt, xq, xqd = pl.pallas_call(
        _vq_kernel,
        grid=(NB,),
        compiler_params=pltpu.CompilerParams(
            dimension_semantics=("parallel",)),
        in_specs=[
            pl.BlockSpec((BB, C, HW), lambda b: (b, 0, 0)),
            pl.BlockSpec((BB, 1, HW), lambda b: (b, 0, 0)),
            pl.BlockSpec((NUM_VECTORS, 1), full),
            pl.BlockSpec((NUM_VECTORS, LATENT_DIM), full),
            pl.BlockSpec((NUM_VECTORS, LATENT_DIM), full),
            pl.BlockSpec((NUM_VECTORS, LATENT_DIM), full),
        ],
        out_specs=[
            pl.BlockSpec((BB, HW, C), lambda b: (b, 0, 0)),
            pl.BlockSpec((BB, HW, C), lambda b: (b, 0, 0)),
            pl.BlockSpec((BB, C, HW), lambda b: (b, 0, 0)),
        ],
        out_shape=[
            jax.ShapeDtypeStruct((B, HW, C), jnp.float32),
            jax.ShapeDtypeStruct((B, HW, C), jnp.float32),
            jax.ShapeDtypeStruct((B, C, HW), jnp.float32),
        ],
    )(x3, xx3, ee2, embedding_weight, e_hi, e_mid_s)

    x_t = xt.reshape(B, H, W, C)
    x_q = xq.reshape(B, H, W, C)
    x_q_detach = xqd.reshape(B, C, H, W)
    return (x_q_detach, x_q, x_t)
